# baseline (speedup 1.0000x reference)
import functools

import jax
import jax.numpy as jnp
from jax import lax
from jax.experimental import pallas as pl
from jax.experimental.pallas import tpu as pltpu
from jax.experimental.pallas import tpu_sc as plsc

HIDDEN = 2048
NE = 8
L = 16
NC, NS = 2, 16
NW = NC * NS
TCH = 16
TG = 4
HC = HIDDEN // L

B_SC = 4096
BT = 1024


def _make_sc_router(base0, b_sc):
    b_per_w = b_sc // NW
    n_chunks = b_per_w // TCH
    assert n_chunks % 2 == 0
    mesh = plsc.VectorSubcoreMesh(core_axis_name="c", subcore_axis_name="s")

    @functools.partial(
        pl.kernel,
        out_type=jax.ShapeDtypeStruct((b_sc * NE // 128, 128), jnp.float32),
        mesh=mesh,
        compiler_params=pltpu.CompilerParams(needs_layout_passes=False),
        scratch_types=[
            pltpu.VMEM((NE, HIDDEN), jnp.float32),
            pltpu.VMEM((2, TCH, HIDDEN), jnp.float32),
            pltpu.VMEM((b_per_w * NE // 128, 128), jnp.float32),
            pltpu.VMEM((TG * NE, L), jnp.float32),
            pltpu.SemaphoreType.DMA,
            pltpu.SemaphoreType.DMA,
            pltpu.SemaphoreType.DMA,
        ],
    )
    def sc_router(x_hbm, w_hbm, out_hbm, w_v, x_v, out_v, acc_v, sem_w, sem0, sem1):
        wid = lax.axis_index("s") * NC + lax.axis_index("c")
        base = base0 + wid * b_per_w
        pltpu.make_async_copy(w_hbm, w_v, sem_w).start()
        pltpu.make_async_copy(w_hbm, w_v, sem_w).wait()

        def start(ci, buf, sem):
            pltpu.make_async_copy(
                x_hbm.at[pl.ds(base + ci * TCH, TCH)], x_v.at[buf], sem
            ).start()

        def wait(ci, buf, sem):
            pltpu.make_async_copy(
                x_hbm.at[pl.ds(base + ci * TCH, TCH)], x_v.at[buf], sem
            ).wait()

        def compute_chunk(ci, buf):
            iot = lax.iota(jnp.int32, L)
            for tg0 in range(0, TCH, TG):
                def hb(c, accs):
                    off = c * L
                    ws = [w_v[e, pl.ds(off, L)] for e in range(NE)]
                    new = []
                    for t in range(TG):
                        xv = x_v[buf, tg0 + t, pl.ds(off, L)]
                        for e in range(NE):
                            new.append(accs[t * NE + e] + xv * ws[e])
                    return tuple(new)

                zero = jnp.zeros((L,), jnp.float32)
                accs = lax.fori_loop(
                    0, HC, hb, tuple(zero for _ in range(TG * NE))
                )
                for r in range(TG * NE):
                    acc_v[r, :] = accs[r]
                for p in range(TG // 2):
                    rows = iot + 16 * p
                    vec = plsc.load_gather(
                        acc_v, [rows, jnp.zeros((L,), jnp.int32)]
                    )
                    for j in range(1, L):
                        vec = vec + plsc.load_gather(
                            acc_v, [rows, jnp.full((L,), j, jnp.int32)]
                        )
                    pi = ci * (TCH // 2) + tg0 // 2 + p
                    out_v[pi >> 3, pl.ds((pi & 7) * L, L)] = vec

        start(0, 0, sem0)

        def pair_body(i, _):
            c0 = 2 * i
            start(c0 + 1, 1, sem1)
            wait(c0, 0, sem0)
            compute_chunk(c0, 0)

            @pl.when(c0 + 2 < n_chunks)
            def _():
                start(c0 + 2, 0, sem0)

            wait(c0 + 1, 1, sem1)
            compute_chunk(c0 + 1, 1)
            return 0

        lax.fori_loop(0, n_chunks // 2, pair_body, 0)

        orows = b_per_w * NE // 128
        pltpu.make_async_copy(
            out_v, out_hbm.at[pl.ds(wid * orows, orows)], sem0
        ).start()
        pltpu.make_async_copy(
            out_v, out_hbm.at[pl.ds(wid * orows, orows)], sem0
        ).wait()

    return sc_router


NBUF = 4


def _make_tc_router(b_full, b_tc):
    assert b_tc % BT == 0
    nch = b_tc // BT

    def body(x_hbm, w_ref, out_hbm, x_v, out_v, xsems, osems):
        def xcopy(ci, b):
            return pltpu.make_async_copy(
                x_hbm.at[pl.ds(ci * BT, BT)], x_v.at[b], xsems.at[b]
            )

        def ocopy(ci, b):
            return pltpu.make_async_copy(
                out_v.at[b], out_hbm.at[pl.ds(ci * BT, BT)], osems.at[b]
            )

        for b in range(NBUF):
            xcopy(b, b).start()

        def group(g, _):
            for b in range(NBUF):
                ci = g * NBUF + b

                @pl.when(ci < nch)
                def _():
                    xcopy(ci, b).wait()

                    @pl.when(g > 0)
                    def _():
                        ocopy(ci - NBUF, b).wait()

                    out_v[b] = lax.dot_general(
                        x_v[b].astype(jnp.bfloat16),
                        w_ref[...].astype(jnp.bfloat16),
                        dimension_numbers=(((1,), (1,)), ((), ())),
                        preferred_element_type=jnp.float32,
                    )
                    ocopy(ci, b).start()

                    @pl.when(ci + NBUF < nch)
                    def _():
                        xcopy(ci + NBUF, b).start()

            return 0

        lax.fori_loop(0, (nch + NBUF - 1) // NBUF, group, 0)
        for b in range(NBUF):
            ocopy(0, b).wait()

    return pl.pallas_call(
        body,
        in_specs=[
            pl.BlockSpec(memory_space=pl.ANY),
            pl.BlockSpec(memory_space=pltpu.VMEM),
        ],
        out_specs=pl.BlockSpec(memory_space=pl.ANY),
        scratch_shapes=[
            pltpu.VMEM((NBUF, BT, HIDDEN), jnp.float32),
            pltpu.VMEM((NBUF, BT, NE), jnp.float32),
            pltpu.SemaphoreType.DMA((NBUF,)),
            pltpu.SemaphoreType.DMA((NBUF,)),
        ],
        out_shape=jax.ShapeDtypeStruct((b_full, NE), jnp.float32),
    )


def kernel(x, weight):
    B = x.shape[0] * x.shape[1]
    xf = x.reshape(B, HIDDEN).astype(jnp.float32)
    wf = weight.astype(jnp.float32)
    return _make_tc_router(B, B)(xf, wf)


# speedup vs baseline: 1.0125x; 1.0125x over previous
import functools

import jax
import jax.numpy as jnp
from jax import lax
from jax.experimental import pallas as pl
from jax.experimental.pallas import tpu as pltpu
from jax.experimental.pallas import tpu_sc as plsc

HIDDEN = 2048
NE = 8
L = 16
NC, NS = 2, 16
NW = NC * NS
TCH = 16
TG = 4
HC = HIDDEN // L

B_SC = 4096
BT = 512


def _make_sc_router(base0, b_sc):
    b_per_w = b_sc // NW
    n_chunks = b_per_w // TCH
    assert n_chunks % 2 == 0
    mesh = plsc.VectorSubcoreMesh(core_axis_name="c", subcore_axis_name="s")

    @functools.partial(
        pl.kernel,
        out_type=jax.ShapeDtypeStruct((b_sc * NE // 128, 128), jnp.float32),
        mesh=mesh,
        compiler_params=pltpu.CompilerParams(needs_layout_passes=False),
        scratch_types=[
            pltpu.VMEM((NE, HIDDEN), jnp.float32),
            pltpu.VMEM((2, TCH, HIDDEN), jnp.float32),
            pltpu.VMEM((b_per_w * NE // 128, 128), jnp.float32),
            pltpu.VMEM((TG * NE, L), jnp.float32),
            pltpu.SemaphoreType.DMA,
            pltpu.SemaphoreType.DMA,
            pltpu.SemaphoreType.DMA,
        ],
    )
    def sc_router(x_hbm, w_hbm, out_hbm, w_v, x_v, out_v, acc_v, sem_w, sem0, sem1):
        wid = lax.axis_index("s") * NC + lax.axis_index("c")
        base = base0 + wid * b_per_w
        pltpu.make_async_copy(w_hbm, w_v, sem_w).start()
        pltpu.make_async_copy(w_hbm, w_v, sem_w).wait()

        def start(ci, buf, sem):
            pltpu.make_async_copy(
                x_hbm.at[pl.ds(base + ci * TCH, TCH)], x_v.at[buf], sem
            ).start()

        def wait(ci, buf, sem):
            pltpu.make_async_copy(
                x_hbm.at[pl.ds(base + ci * TCH, TCH)], x_v.at[buf], sem
            ).wait()

        def compute_chunk(ci, buf):
            iot = lax.iota(jnp.int32, L)
            for tg0 in range(0, TCH, TG):
                def hb(c, accs):
                    off = c * L
                    ws = [w_v[e, pl.ds(off, L)] for e in range(NE)]
                    new = []
                    for t in range(TG):
                        xv = x_v[buf, tg0 + t, pl.ds(off, L)]
                        for e in range(NE):
                            new.append(accs[t * NE + e] + xv * ws[e])
                    return tuple(new)

                zero = jnp.zeros((L,), jnp.float32)
                accs = lax.fori_loop(
                    0, HC, hb, tuple(zero for _ in range(TG * NE))
                )
                for r in range(TG * NE):
                    acc_v[r, :] = accs[r]
                for p in range(TG // 2):
                    rows = iot + 16 * p
                    vec = plsc.load_gather(
                        acc_v, [rows, jnp.zeros((L,), jnp.int32)]
                    )
                    for j in range(1, L):
                        vec = vec + plsc.load_gather(
                            acc_v, [rows, jnp.full((L,), j, jnp.int32)]
                        )
                    pi = ci * (TCH // 2) + tg0 // 2 + p
                    out_v[pi >> 3, pl.ds((pi & 7) * L, L)] = vec

        start(0, 0, sem0)

        def pair_body(i, _):
            c0 = 2 * i
            start(c0 + 1, 1, sem1)
            wait(c0, 0, sem0)
            compute_chunk(c0, 0)

            @pl.when(c0 + 2 < n_chunks)
            def _():
                start(c0 + 2, 0, sem0)

            wait(c0 + 1, 1, sem1)
            compute_chunk(c0 + 1, 1)
            return 0

        lax.fori_loop(0, n_chunks // 2, pair_body, 0)

        orows = b_per_w * NE // 128
        pltpu.make_async_copy(
            out_v, out_hbm.at[pl.ds(wid * orows, orows)], sem0
        ).start()
        pltpu.make_async_copy(
            out_v, out_hbm.at[pl.ds(wid * orows, orows)], sem0
        ).wait()

    return sc_router


NBUF = 8


def _make_tc_router(b_full, b_tc):
    assert b_tc % BT == 0
    nch = b_tc // BT

    def body(x_hbm, w_ref, out_hbm, x_v, out_v, xsems, osems):
        def xcopy(ci, b):
            return pltpu.make_async_copy(
                x_hbm.at[pl.ds(ci * BT, BT)], x_v.at[b], xsems.at[b]
            )

        def ocopy(ci, b):
            return pltpu.make_async_copy(
                out_v.at[b], out_hbm.at[pl.ds(ci * BT, BT)], osems.at[b]
            )

        for b in range(NBUF):
            xcopy(b, b).start()

        def group(g, _):
            for b in range(NBUF):
                ci = g * NBUF + b

                @pl.when(ci < nch)
                def _():
                    xcopy(ci, b).wait()

                    @pl.when(g > 0)
                    def _():
                        ocopy(ci - NBUF, b).wait()

                    out_v[b] = lax.dot_general(
                        x_v[b].astype(jnp.bfloat16),
                        w_ref[...].astype(jnp.bfloat16),
                        dimension_numbers=(((1,), (1,)), ((), ())),
                        preferred_element_type=jnp.float32,
                    )
                    ocopy(ci, b).start()

                    @pl.when(ci + NBUF < nch)
                    def _():
                        xcopy(ci + NBUF, b).start()

            return 0

        lax.fori_loop(0, (nch + NBUF - 1) // NBUF, group, 0)
        for b in range(NBUF):
            ocopy(0, b).wait()

    return pl.pallas_call(
        body,
        in_specs=[
            pl.BlockSpec(memory_space=pl.ANY),
            pl.BlockSpec(memory_space=pltpu.VMEM),
        ],
        out_specs=pl.BlockSpec(memory_space=pl.ANY),
        scratch_shapes=[
            pltpu.VMEM((NBUF, BT, HIDDEN), jnp.float32),
            pltpu.VMEM((NBUF, BT, NE), jnp.float32),
            pltpu.SemaphoreType.DMA((NBUF,)),
            pltpu.SemaphoreType.DMA((NBUF,)),
        ],
        out_shape=jax.ShapeDtypeStruct((b_full, NE), jnp.float32),
    )


def kernel(x, weight):
    B = x.shape[0] * x.shape[1]
    xf = x.reshape(B, HIDDEN).astype(jnp.float32)
    wf = weight.astype(jnp.float32)
    return _make_tc_router(B, B)(xf, wf)


# speedup vs baseline: 1.2050x; 1.1901x over previous
import functools

import jax
import jax.numpy as jnp
from jax import lax
from jax.experimental import pallas as pl
from jax.experimental.pallas import tpu as pltpu
from jax.experimental.pallas import tpu_sc as plsc

HIDDEN = 2048
NE = 8
L = 16
NC, NS = 2, 16
NW = NC * NS
TCH = 16
TG = 4
HC = HIDDEN // L

B_SC = 4096
BT = 512


def _make_sc_router(base0, b_sc):
    b_per_w = b_sc // NW
    n_chunks = b_per_w // TCH
    assert n_chunks % 2 == 0
    mesh = plsc.VectorSubcoreMesh(core_axis_name="c", subcore_axis_name="s")

    @functools.partial(
        pl.kernel,
        out_type=jax.ShapeDtypeStruct((b_sc * NE // 128, 128), jnp.float32),
        mesh=mesh,
        compiler_params=pltpu.CompilerParams(needs_layout_passes=False),
        scratch_types=[
            pltpu.VMEM((NE, HIDDEN), jnp.float32),
            pltpu.VMEM((2, TCH, HIDDEN), jnp.float32),
            pltpu.VMEM((b_per_w * NE // 128, 128), jnp.float32),
            pltpu.VMEM((TG * NE, L), jnp.float32),
            pltpu.SemaphoreType.DMA,
            pltpu.SemaphoreType.DMA,
            pltpu.SemaphoreType.DMA,
        ],
    )
    def sc_router(x_hbm, w_hbm, out_hbm, w_v, x_v, out_v, acc_v, sem_w, sem0, sem1):
        wid = lax.axis_index("s") * NC + lax.axis_index("c")
        base = base0 + wid * b_per_w
        pltpu.make_async_copy(w_hbm, w_v, sem_w).start()
        pltpu.make_async_copy(w_hbm, w_v, sem_w).wait()

        def start(ci, buf, sem):
            pltpu.make_async_copy(
                x_hbm.at[pl.ds(base + ci * TCH, TCH)], x_v.at[buf], sem
            ).start()

        def wait(ci, buf, sem):
            pltpu.make_async_copy(
                x_hbm.at[pl.ds(base + ci * TCH, TCH)], x_v.at[buf], sem
            ).wait()

        def compute_chunk(ci, buf):
            iot = lax.iota(jnp.int32, L)
            for tg0 in range(0, TCH, TG):
                def hb(c, accs):
                    off = c * L
                    ws = [w_v[e, pl.ds(off, L)] for e in range(NE)]
                    new = []
                    for t in range(TG):
                        xv = x_v[buf, tg0 + t, pl.ds(off, L)]
                        for e in range(NE):
                            new.append(accs[t * NE + e] + xv * ws[e])
                    return tuple(new)

                zero = jnp.zeros((L,), jnp.float32)
                accs = lax.fori_loop(
                    0, HC, hb, tuple(zero for _ in range(TG * NE))
                )
                for r in range(TG * NE):
                    acc_v[r, :] = accs[r]
                for p in range(TG // 2):
                    rows = iot + 16 * p
                    vec = plsc.load_gather(
                        acc_v, [rows, jnp.zeros((L,), jnp.int32)]
                    )
                    for j in range(1, L):
                        vec = vec + plsc.load_gather(
                            acc_v, [rows, jnp.full((L,), j, jnp.int32)]
                        )
                    pi = ci * (TCH // 2) + tg0 // 2 + p
                    out_v[pi >> 3, pl.ds((pi & 7) * L, L)] = vec

        start(0, 0, sem0)

        def pair_body(i, _):
            c0 = 2 * i
            start(c0 + 1, 1, sem1)
            wait(c0, 0, sem0)
            compute_chunk(c0, 0)

            @pl.when(c0 + 2 < n_chunks)
            def _():
                start(c0 + 2, 0, sem0)

            wait(c0 + 1, 1, sem1)
            compute_chunk(c0 + 1, 1)
            return 0

        lax.fori_loop(0, n_chunks // 2, pair_body, 0)

        orows = b_per_w * NE // 128
        pltpu.make_async_copy(
            out_v, out_hbm.at[pl.ds(wid * orows, orows)], sem0
        ).start()
        pltpu.make_async_copy(
            out_v, out_hbm.at[pl.ds(wid * orows, orows)], sem0
        ).wait()

    return sc_router


NBUF = 8


def _make_tc_router(b_full, b_tc):
    assert b_tc % BT == 0
    nch = b_tc // BT

    def body(x_hbm, w_ref, out_hbm, x_v, out_v, xsems, osems):
        def xcopy(ci, b):
            return pltpu.make_async_copy(
                x_hbm.at[pl.ds(ci * BT, BT)], x_v.at[b], xsems.at[b]
            )

        def ocopy(ci, b):
            return pltpu.make_async_copy(
                out_v.at[b], out_hbm.at[:, pl.ds(ci * BT, BT)], osems.at[b]
            )

        for b in range(NBUF):
            xcopy(b, b).start()

        def group(g, _):
            for b in range(NBUF):
                ci = g * NBUF + b

                @pl.when(ci < nch)
                def _():
                    xcopy(ci, b).wait()

                    @pl.when(g > 0)
                    def _():
                        ocopy(ci - NBUF, b).wait()

                    out_v[b] = lax.dot_general(
                        w_ref[...].astype(jnp.bfloat16),
                        x_v[b].astype(jnp.bfloat16),
                        dimension_numbers=(((1,), (1,)), ((), ())),
                        preferred_element_type=jnp.float32,
                    )
                    ocopy(ci, b).start()

                    @pl.when(ci + NBUF < nch)
                    def _():
                        xcopy(ci + NBUF, b).start()

            return 0

        lax.fori_loop(0, (nch + NBUF - 1) // NBUF, group, 0)
        for b in range(NBUF):
            ocopy(0, b).wait()

    return pl.pallas_call(
        body,
        in_specs=[
            pl.BlockSpec(memory_space=pl.ANY),
            pl.BlockSpec(memory_space=pltpu.VMEM),
        ],
        out_specs=pl.BlockSpec(memory_space=pl.ANY),
        scratch_shapes=[
            pltpu.VMEM((NBUF, BT, HIDDEN), jnp.float32),
            pltpu.VMEM((NBUF, NE, BT), jnp.float32),
            pltpu.SemaphoreType.DMA((NBUF,)),
            pltpu.SemaphoreType.DMA((NBUF,)),
        ],
        out_shape=jax.ShapeDtypeStruct((NE, b_full), jnp.float32),
    )


def kernel(x, weight):
    B = x.shape[0] * x.shape[1]
    xf = x.reshape(B, HIDDEN).astype(jnp.float32)
    wf = weight.astype(jnp.float32)
    return _make_tc_router(B, B)(xf, wf).T
